# baseline (speedup 1.0000x reference)
import functools

import jax
import jax.numpy as jnp
from jax import lax
from jax.experimental import pallas as pl
from jax.experimental.pallas import tpu as pltpu
from jax.experimental.pallas import tpu_sc as plsc

N_FIELDS = 26
VOCAB = 100000
EMBED = 32
BATCH = 16384

NC = 2
NS = 16
NW = NC * NS

VROWS = 100096
VFULL = 99328
VTAIL = 99968
NTAIL = N_FIELDS * (VOCAB + 1 - VTAIL) * EMBED

TOT = BATCH * N_FIELDS
PER_W = TOT // NW
CHUNK = 1664
NCHUNK = PER_W // CHUNK

CHUNKS_PER_F = 98
NQ = N_FIELDS * CHUNKS_PER_F

_mesh = plsc.VectorSubcoreMesh(core_axis_name="c", subcore_axis_name="s")


@functools.partial(
    pl.kernel,
    mesh=_mesh,
    out_type=jax.ShapeDtypeStruct((N_FIELDS * VROWS * EMBED,), jnp.float32),
    scratch_types=[
        pltpu.VMEM((8, 1024), jnp.float32),
        pltpu.VMEM((8, 1024), jnp.float32),
        pltpu.VMEM((8, 1024), jnp.float32),
        pltpu.VMEM((8, 1024), jnp.float32),
        pltpu.VMEM((32768,), jnp.float32),
        pltpu.SemaphoreType.DMA,
    ],
    compiler_params=pltpu.CompilerParams(needs_layout_passes=False),
)
def _detranspose_kernel(tabt_hbm, tail_hbm, out_hbm, s0, s1, s2, s3, outv, sem):
    wid = lax.axis_index("s") * NC + lax.axis_index("c")
    staged = (s0, s1, s2, s3)
    lanes32 = lax.iota(jnp.int32, 16) * 32

    def do_chunk(f, v0, m):
        copies = [
            pltpu.async_copy(
                tabt_hbm.at[f, pl.ds(8 * a, 8), pl.ds(v0, 128 * m)],
                staged[a].at[:, pl.ds(0, 128 * m)],
                sem,
            )
            for a in range(4)
        ]
        for cp in copies:
            cp.wait()

        def scat(k, _):
            k512 = k * 512
            for a in range(4):
                for er in range(8):
                    vec = staged[a][er, pl.ds(k * 16, 16)]
                    idxv = lanes32 + (k512 + (8 * a + er))
                    plsc.store_scatter(outv, [idxv], vec)
            return 0

        lax.fori_loop(0, 8 * m, scat, 0)

        pltpu.sync_copy(
            outv.at[pl.ds(0, 4096 * m)],
            out_hbm.at[pl.ds((f * VROWS + v0) * EMBED, 4096 * m)],
        )

    def body(t, _):
        q = wid + NW * t

        @pl.when(q < NQ)
        def _():
            f = q // CHUNKS_PER_F
            j = q % CHUNKS_PER_F

            @pl.when(j < 97)
            def _():
                do_chunk(f, 1024 * j, 8)

            @pl.when(j == 97)
            def _():
                do_chunk(f, VFULL, 5)

        return 0

    lax.fori_loop(0, (NQ + NW - 1) // NW, body, 0)

    @pl.when(wid < N_FIELDS)
    def _():
        n = (VOCAB + 1 - VTAIL) * EMBED
        pltpu.sync_copy(tail_hbm.at[pl.ds(wid * n, n)], outv.at[pl.ds(0, n)])
        pltpu.sync_copy(
            outv.at[pl.ds(0, n)],
            out_hbm.at[pl.ds((wid * VROWS + VTAIL) * EMBED, n)],
        )


@functools.partial(
    pl.kernel,
    mesh=_mesh,
    out_type=jax.ShapeDtypeStruct((TOT, EMBED), jnp.float32),
    scratch_types=[
        pltpu.VMEM((PER_W,), jnp.int32),
        pltpu.VMEM((CHUNK, EMBED), jnp.float32),
        pltpu.SemaphoreType.DMA,
    ],
    compiler_params=pltpu.CompilerParams(use_tc_tiling_on_sc=False),
)
def _gather_kernel(xg_hbm, tab_hbm, out_hbm, idx_v, rows_v, sem):
    wid = lax.axis_index("s") * NC + lax.axis_index("c")
    base = wid * PER_W

    pltpu.sync_copy(xg_hbm.at[pl.ds(base, PER_W)], idx_v)

    lanes = lax.iota(jnp.int32, 16)

    def add_body(r, _):
        for j in range(8):
            s = r * 128 + j * 16
            p0 = base + s
            field = lax.rem(p0 + lanes, N_FIELDS)
            idx_v[pl.ds(s, 16)] = idx_v[pl.ds(s, 16)] + field * VROWS
        return 0

    lax.fori_loop(0, PER_W // 128, add_body, 0)

    def chunk_body(c, _):
        kb = c * CHUNK
        pltpu.async_copy(tab_hbm.at[idx_v.at[pl.ds(kb, CHUNK)]], rows_v, sem).wait()
        pltpu.sync_copy(rows_v, out_hbm.at[pl.ds(base + kb, CHUNK)])
        return 0

    lax.fori_loop(0, NCHUNK, chunk_body, 0)


def kernel(x, tables):
    xg = x.reshape(-1).astype(jnp.int32)
    tabt = jnp.transpose(tables, (0, 2, 1))
    tail = tables[:, VTAIL:, :].reshape(-1)
    flat = _detranspose_kernel(tabt, tail)
    tab = flat.reshape(N_FIELDS * VROWS, EMBED)
    out = _gather_kernel(xg, tab)
    return out.reshape(BATCH, N_FIELDS * EMBED)


# speedup vs baseline: 1.0917x; 1.0917x over previous
import functools

import jax
import jax.numpy as jnp
from jax import lax
from jax.experimental import pallas as pl
from jax.experimental.pallas import tpu as pltpu
from jax.experimental.pallas import tpu_sc as plsc

N_FIELDS = 26
VOCAB = 100000
EMBED = 32
BATCH = 16384

NC = 2
NS = 16
NW = NC * NS

VROWS = 100096
VFULL = 99328
NREST = (VOCAB + 1 - VFULL) * EMBED

TOT = BATCH * N_FIELDS
PER_W = TOT // NW
CHUNK = 1664
NCHUNK = PER_W // CHUNK

CHUNKS_PER_F = 97
NQ = N_FIELDS * CHUNKS_PER_F
TW = (NQ + NW - 1) // NW
OUTW = 1024 * EMBED

_mesh = plsc.VectorSubcoreMesh(core_axis_name="c", subcore_axis_name="s")


@functools.partial(
    pl.kernel,
    mesh=_mesh,
    out_type=jax.ShapeDtypeStruct((N_FIELDS * VROWS * EMBED,), jnp.float32),
    scratch_types=[
        pltpu.VMEM((8, 1024), jnp.float32),
        pltpu.VMEM((8, 1024), jnp.float32),
        pltpu.VMEM((8, 1024), jnp.float32),
        pltpu.VMEM((8, 1024), jnp.float32),
        pltpu.VMEM((8, 1024), jnp.float32),
        pltpu.VMEM((8, 1024), jnp.float32),
        pltpu.VMEM((8, 1024), jnp.float32),
        pltpu.VMEM((8, 1024), jnp.float32),
        pltpu.VMEM((OUTW,), jnp.float32),
        pltpu.SemaphoreType.DMA,
        pltpu.SemaphoreType.DMA,
    ],
    compiler_params=pltpu.CompilerParams(needs_layout_passes=False),
)
def _detranspose_kernel(
    tabt_hbm, rest_hbm, out_hbm, s0, s1, s2, s3, s4, s5, s6, s7, outv, sem_i, sem_o
):
    wid = lax.axis_index("s") * NC + lax.axis_index("c")
    slots = ((s0, s1, s2, s3), (s4, s5, s6, s7))
    lanes32 = lax.iota(jnp.int32, 16) * 32

    def in_copies(q, bufs):
        f = q // CHUNKS_PER_F
        v0 = 1024 * (q % CHUNKS_PER_F)
        return [
            pltpu.make_async_copy(
                tabt_hbm.at[f, pl.ds(8 * a, 8), pl.ds(v0, 1024)], bufs[a], sem_i
            )
            for a in range(4)
        ]

    def compute(bufs):
        def scat(k, _):
            k512 = k * 512
            for a in range(4):
                for er in range(8):
                    vec = bufs[a][er, pl.ds(k * 16, 16)]
                    idxv = lanes32 + (k512 + (8 * a + er))
                    plsc.store_scatter(outv, [idxv], vec)
            return 0

        lax.fori_loop(0, 64, scat, 0)

    def out_drain():
        pltpu.make_async_copy(outv, out_hbm.at[pl.ds(0, OUTW)], sem_o).wait()

    def phase(t, slot):
        q = wid + NW * t

        @pl.when(q < NQ)
        def _():
            nq = q + NW

            @pl.when(nq < NQ)
            def _():
                for cp in in_copies(nq, slots[1 - slot]):
                    cp.start()

            for cp in in_copies(q, slots[slot]):
                cp.wait()

            @pl.when(t > 0)
            def _():
                out_drain()

            compute(slots[slot])
            f = q // CHUNKS_PER_F
            v0 = 1024 * (q % CHUNKS_PER_F)
            pltpu.make_async_copy(
                outv, out_hbm.at[pl.ds((f * VROWS + v0) * EMBED, OUTW)], sem_o
            ).start()

    for cp in in_copies(wid, slots[0]):
        cp.start()

    def body(t2, _):
        phase(2 * t2, 0)
        phase(2 * t2 + 1, 1)
        return 0

    lax.fori_loop(0, (TW + 1) // 2, body, 0)
    out_drain()

    @pl.when(wid < N_FIELDS)
    def _():
        pltpu.sync_copy(rest_hbm.at[pl.ds(wid * NREST, NREST)], outv.at[pl.ds(0, NREST)])
        pltpu.sync_copy(
            outv.at[pl.ds(0, NREST)],
            out_hbm.at[pl.ds((wid * VROWS + VFULL) * EMBED, NREST)],
        )


@functools.partial(
    pl.kernel,
    mesh=_mesh,
    out_type=jax.ShapeDtypeStruct((TOT, EMBED), jnp.float32),
    scratch_types=[
        pltpu.VMEM((PER_W,), jnp.int32),
        pltpu.VMEM((CHUNK, EMBED), jnp.float32),
        pltpu.SemaphoreType.DMA,
    ],
    compiler_params=pltpu.CompilerParams(use_tc_tiling_on_sc=False),
)
def _gather_kernel(xg_hbm, tab_hbm, out_hbm, idx_v, rows_v, sem):
    wid = lax.axis_index("s") * NC + lax.axis_index("c")
    base = wid * PER_W

    pltpu.sync_copy(xg_hbm.at[pl.ds(base, PER_W)], idx_v)

    lanes = lax.iota(jnp.int32, 16)

    def add_body(r, _):
        for j in range(8):
            s = r * 128 + j * 16
            p0 = base + s
            field = lax.rem(p0 + lanes, N_FIELDS)
            idx_v[pl.ds(s, 16)] = idx_v[pl.ds(s, 16)] + field * VROWS
        return 0

    lax.fori_loop(0, PER_W // 128, add_body, 0)

    def chunk_body(c, _):
        kb = c * CHUNK
        pltpu.async_copy(tab_hbm.at[idx_v.at[pl.ds(kb, CHUNK)]], rows_v, sem).wait()
        pltpu.sync_copy(rows_v, out_hbm.at[pl.ds(base + kb, CHUNK)])
        return 0

    lax.fori_loop(0, NCHUNK, chunk_body, 0)


def kernel(x, tables):
    xg = x.reshape(-1).astype(jnp.int32)
    tabt = jnp.transpose(tables, (0, 2, 1))
    rest = tables[:, VFULL:, :].reshape(-1)
    flat = _detranspose_kernel(tabt, rest)
    tab = flat.reshape(N_FIELDS * VROWS, EMBED)
    out = _gather_kernel(xg, tab)
    return out.reshape(BATCH, N_FIELDS * EMBED)
